# Initial kernel scaffold; baseline (speedup 1.0000x reference)
import jax
import jax.numpy as jnp
from jax.experimental import pallas as pl


def kernel(x, assessment, index):
    raise NotImplementedError("write your pallas kernel here")


# speedup vs baseline: 1.0349x; 1.0349x over previous
import functools

import jax
import jax.numpy as jnp
from jax import lax
from jax.experimental import pallas as pl
from jax.experimental.pallas import tpu as pltpu
from jax.experimental.pallas import tpu_sc as plsc

_B, _N, _K = 64, 32768, 2048
_NC, _NS = 2, 16
_NW = _NC * _NS
_ROWS_PER_W = _B // _NW
_L = 16
_STEPS = _K // _L


def _gather_body(x_hbm, idx_hbm, out_hbm, x_v, idx_v, out_v):
    wid = lax.axis_index("s") * _NC + lax.axis_index("c")
    for r in range(_ROWS_PER_W):
        row = wid * _ROWS_PER_W + r
        pltpu.sync_copy(x_hbm.at[row], x_v)
        pltpu.sync_copy(idx_hbm.at[row], idx_v)

        def step(i, carry):
            iv = idx_v[pl.ds(i * _L, _L)]
            out_v[pl.ds(i * _L, _L)] = plsc.load_gather(x_v, [iv])
            return carry

        lax.fori_loop(0, _STEPS, step, 0, unroll=4)
        pltpu.sync_copy(out_v, out_hbm.at[row])


@jax.jit
def _run(x, idx):
    mesh = plsc.VectorSubcoreMesh(core_axis_name="c", subcore_axis_name="s")
    f = pl.kernel(
        _gather_body,
        out_type=jax.ShapeDtypeStruct((_B, _K), jnp.float32),
        mesh=mesh,
        scratch_types=[
            pltpu.VMEM((_N,), jnp.float32),
            pltpu.VMEM((_K,), jnp.int32),
            pltpu.VMEM((_K,), jnp.float32),
        ],
        compiler_params=pltpu.CompilerParams(needs_layout_passes=False),
    )
    return f(x, idx)


def kernel(x, assessment, index):
    del assessment
    return _run(x, index.astype(jnp.int32))


# speedup vs baseline: 1.1175x; 1.0798x over previous
import functools

import jax
import jax.numpy as jnp
from jax import lax
from jax.experimental import pallas as pl
from jax.experimental.pallas import tpu as pltpu
from jax.experimental.pallas import tpu_sc as plsc

_B, _N, _K = 64, 32768, 2048
_NC, _NS = 2, 16
_NW = _NC * _NS
_ROWS_PER_W = _B // _NW
_L = 16
_STEPS = _K // _L


def _gather_body(x_hbm, idx_hbm, out_hbm,
                 x0_v, x1_v, i0_v, i1_v, o0_v, o1_v,
                 sem_a, sem_b, sem_o):
    wid = lax.axis_index("s") * _NC + lax.axis_index("c")
    row0 = wid * _ROWS_PER_W
    row1 = row0 + 1

    dx0 = pltpu.async_copy(x_hbm.at[row0], x0_v, sem_a)
    di0 = pltpu.async_copy(idx_hbm.at[row0], i0_v, sem_a)
    dx1 = pltpu.async_copy(x_hbm.at[row1], x1_v, sem_b)
    di1 = pltpu.async_copy(idx_hbm.at[row1], i1_v, sem_b)

    def gather_row(x_v, idx_v, out_v):
        def step(i, carry):
            iv = idx_v[pl.ds(i * _L, _L)]
            out_v[pl.ds(i * _L, _L)] = plsc.load_gather(x_v, [iv])
            return carry

        lax.fori_loop(0, _STEPS, step, 0, unroll=8)

    dx0.wait()
    di0.wait()
    gather_row(x0_v, i0_v, o0_v)
    do0 = pltpu.async_copy(o0_v, out_hbm.at[row0], sem_o)
    dx1.wait()
    di1.wait()
    gather_row(x1_v, i1_v, o1_v)
    do1 = pltpu.async_copy(o1_v, out_hbm.at[row1], sem_o)
    do0.wait()
    do1.wait()


@jax.jit
def _run(x, idx):
    mesh = plsc.VectorSubcoreMesh(core_axis_name="c", subcore_axis_name="s")
    f = pl.kernel(
        _gather_body,
        out_type=jax.ShapeDtypeStruct((_B, _K), jnp.float32),
        mesh=mesh,
        scratch_types=[
            pltpu.VMEM((_N,), jnp.float32),
            pltpu.VMEM((_N,), jnp.float32),
            pltpu.VMEM((_K,), jnp.int32),
            pltpu.VMEM((_K,), jnp.int32),
            pltpu.VMEM((_K,), jnp.float32),
            pltpu.VMEM((_K,), jnp.float32),
            pltpu.SemaphoreType.DMA,
            pltpu.SemaphoreType.DMA,
            pltpu.SemaphoreType.DMA,
        ],
        compiler_params=pltpu.CompilerParams(needs_layout_passes=False),
    )
    return f(x, idx)


def kernel(x, assessment, index):
    del assessment
    return _run(x, index.astype(jnp.int32))
